# Initial kernel scaffold; baseline (speedup 1.0000x reference)
import jax
import jax.numpy as jnp
from jax.experimental import pallas as pl


def kernel(current_position_ids, past_position_ids, pos_emb_weight):
    raise NotImplementedError("write your pallas kernel here")


# speedup vs baseline: 5.5742x; 5.5742x over previous
import functools

import jax
import jax.numpy as jnp
from jax import lax
from jax.experimental import pallas as pl
from jax.experimental.pallas import tpu as pltpu
from jax.experimental.pallas import tpu_sc as plsc

D = 128
B = 16384
NC, NS = 2, 16
NW = NC * NS
PER_W = B // NW
CHUNK = 128
NCHUNK = PER_W // CHUNK


def _norm_body(w_ref, o_ref):
    w = w_ref[...]
    sq = w * w
    left = jnp.concatenate([sq[:, 1:], sq[:, :1]], axis=1)
    right = jnp.concatenate([sq[:, :1], sq[:, :-1]], axis=1)
    lane = lax.broadcasted_iota(jnp.int32, w.shape, 1)
    pair = sq + jnp.where(lane % 2 == 0, left, right)
    o_ref[...] = w / jnp.sqrt(pair)


def _normalize(w):
    return pl.pallas_call(
        _norm_body,
        out_shape=jax.ShapeDtypeStruct(w.shape, w.dtype),
    )(w)


def _gather_body(table, idx_a, idx_b, out_a, out_b, idx_v, rows, sem):
    wid = lax.axis_index("s") * NC + lax.axis_index("c")
    for idx_hbm, out_hbm in ((idx_a, out_a), (idx_b, out_b)):
        pltpu.sync_copy(idx_hbm.at[pl.ds(wid * NCHUNK, NCHUNK)], idx_v)
        copies = [
            pltpu.make_async_copy(
                table.at[idx_v.at[j]],
                rows.at[pl.ds(j * CHUNK, CHUNK)],
                sem,
            )
            for j in range(NCHUNK)
        ]
        for c in copies:
            c.start()
        for c in copies:
            c.wait()
        pltpu.sync_copy(rows, out_hbm.at[pl.ds(wid * PER_W, PER_W)])


def kernel(current_position_ids, past_position_ids, pos_emb_weight):
    w = _normalize(pos_emb_weight)
    idx_a = current_position_ids.reshape(B // CHUNK, CHUNK)
    idx_b = past_position_ids.reshape(B // CHUNK, CHUNK)

    mesh = plsc.VectorSubcoreMesh(
        core_axis_name="c", subcore_axis_name="s",
        num_cores=NC, num_subcores=NS,
    )
    gather = pl.kernel(
        _gather_body,
        out_type=[
            jax.ShapeDtypeStruct((B, D), jnp.float32),
            jax.ShapeDtypeStruct((B, D), jnp.float32),
        ],
        mesh=mesh,
        scratch_types=[
            pltpu.VMEM((NCHUNK, CHUNK), jnp.int32),
            pltpu.VMEM((PER_W, D), jnp.float32),
            pltpu.SemaphoreType.DMA,
        ],
    )
    out_a, out_b = gather(w, idx_a, idx_b)

    shp = current_position_ids.shape
    f_a = out_a.reshape(shp[0], shp[1], D // 2, 2)[:, None]
    f_b = out_b.reshape(shp[0], shp[1], D // 2, 2)[:, None]
    return (f_a, f_b)
